# baseline (speedup 1.0000x reference)
import functools

import jax
import jax.numpy as jnp
from jax import lax
from jax.experimental import pallas as pl
from jax.experimental.pallas import tpu as pltpu
from jax.experimental.pallas import tpu_sc as plsc

IDX_NUM = 1000000
LATENT_DIM = 16
BATCH = 16384
HIST = 50

NC = 2
NS = 16
NW = NC * NS
BAGS_PER_W = BATCH // NW
CHUNK_BAGS = 64
N_CHUNKS = BAGS_PER_W // CHUNK_BAGS
ROWS_PER_CHUNK = CHUNK_BAGS * HIST
IDX_SLICES = ROWS_PER_CHUNK // 128


def _sc_body(x_hbm, v_hbm, s_hbm, out_hbm, idx_v, rows_v, out_v, s_v, sem):
    wid = lax.axis_index("s") * NC + lax.axis_index("c")

    pltpu.sync_copy(s_hbm, s_v)
    sval = s_v[...]
    t = 1.0 / (1.0 + jnp.exp(-sval))
    nt = -t

    pltpu.sync_copy(x_hbm.at[wid], idx_v)

    def chunk_body(c, carry):
        pltpu.async_copy(
            v_hbm.at[idx_v.at[pl.ds(c * ROWS_PER_CHUNK, ROWS_PER_CHUNK)]],
            rows_v,
            sem,
        ).wait()

        def bag_body(b, carry2):
            base = b * HIST
            acc = jnp.zeros((LATENT_DIM,), jnp.float32)
            for l in range(HIST):
                r = rows_v[base + l]
                acc = acc + (r - jnp.minimum(jnp.maximum(r, nt), t))
            out_v[b] = acc
            return carry2

        lax.fori_loop(0, CHUNK_BAGS, bag_body, 0)

        bag_base = wid * BAGS_PER_W + c * CHUNK_BAGS
        pltpu.sync_copy(out_v, out_hbm.at[pl.ds(bag_base, CHUNK_BAGS)])
        return carry

    lax.fori_loop(0, N_CHUNKS, chunk_body, 0)


@jax.jit
def _run(x2d, v, s):
    mesh = plsc.VectorSubcoreMesh(core_axis_name="c", subcore_axis_name="s")
    return pl.kernel(
        _sc_body,
        out_type=jax.ShapeDtypeStruct((BATCH, LATENT_DIM), jnp.float32),
        mesh=mesh,
        compiler_params=pltpu.CompilerParams(use_tc_tiling_on_sc=False),
        scratch_types=[
            pltpu.VMEM((BAGS_PER_W * HIST,), jnp.int32),
            pltpu.VMEM((ROWS_PER_CHUNK, LATENT_DIM), jnp.float32),
            pltpu.VMEM((CHUNK_BAGS, LATENT_DIM), jnp.float32),
            pltpu.VMEM((LATENT_DIM,), jnp.float32),
            pltpu.SemaphoreType.DMA,
        ],
    )(x2d, v, s)


def kernel(x, v, s):
    x2d = x.reshape(NW, BAGS_PER_W * HIST).astype(jnp.int32)
    return _run(x2d, v, s)


# speedup vs baseline: 1.0440x; 1.0440x over previous
import functools

import jax
import jax.numpy as jnp
from jax import lax
from jax.experimental import pallas as pl
from jax.experimental.pallas import tpu as pltpu
from jax.experimental.pallas import tpu_sc as plsc

IDX_NUM = 1000000
LATENT_DIM = 16
BATCH = 16384
HIST = 50

NC = 2
NS = 16
NW = NC * NS
BAGS_PER_W = BATCH // NW
CHUNK_BAGS = 64
N_CHUNKS = BAGS_PER_W // CHUNK_BAGS
ROWS_PER_CHUNK = CHUNK_BAGS * HIST
IDX_SLICES = ROWS_PER_CHUNK // 128


def _sc_body(x_hbm, v_hbm, s_hbm, out_hbm, idx_v, rows_v, out_v, s_v, sem):
    wid = lax.axis_index("s") * NC + lax.axis_index("c")

    pltpu.sync_copy(s_hbm, s_v)
    sval = s_v[...]
    t = 1.0 / (1.0 + jnp.exp(-sval))
    nt = -t

    pltpu.sync_copy(x_hbm.at[wid], idx_v)

    def issue(c):
        return pltpu.async_copy(
            v_hbm.at[idx_v.at[pl.ds(c * ROWS_PER_CHUNK, ROWS_PER_CHUNK)]],
            rows_v.at[c % 2],
            sem,
        )

    h = issue(0)
    for c in range(N_CHUNKS):
        h_next = issue(c + 1) if c + 1 < N_CHUNKS else None
        h.wait()
        buf = c % 2

        def bag_body(b, carry2):
            base = b * HIST
            acc = jnp.zeros((LATENT_DIM,), jnp.float32)
            for l in range(HIST):
                r = rows_v[buf, base + l]
                acc = acc + (r - jnp.minimum(jnp.maximum(r, nt), t))
            out_v[b] = acc
            return carry2

        lax.fori_loop(0, CHUNK_BAGS, bag_body, 0)

        bag_base = wid * BAGS_PER_W + c * CHUNK_BAGS
        pltpu.sync_copy(out_v, out_hbm.at[pl.ds(bag_base, CHUNK_BAGS)])
        h = h_next


@jax.jit
def _run(x2d, v, s):
    mesh = plsc.VectorSubcoreMesh(core_axis_name="c", subcore_axis_name="s")
    return pl.kernel(
        _sc_body,
        out_type=jax.ShapeDtypeStruct((BATCH, LATENT_DIM), jnp.float32),
        mesh=mesh,
        compiler_params=pltpu.CompilerParams(use_tc_tiling_on_sc=False),
        scratch_types=[
            pltpu.VMEM((BAGS_PER_W * HIST,), jnp.int32),
            pltpu.VMEM((2, ROWS_PER_CHUNK, LATENT_DIM), jnp.float32),
            pltpu.VMEM((CHUNK_BAGS, LATENT_DIM), jnp.float32),
            pltpu.VMEM((LATENT_DIM,), jnp.float32),
            pltpu.SemaphoreType.DMA,
        ],
    )(x2d, v, s)


def kernel(x, v, s):
    x2d = x.reshape(NW, BAGS_PER_W * HIST).astype(jnp.int32)
    return _run(x2d, v, s)
